# Initial kernel scaffold; baseline (speedup 1.0000x reference)
import jax
import jax.numpy as jnp
from jax.experimental import pallas as pl


def kernel(objs, triples, obj_table, pred_table, W1, b1, W2, b2, W3, b3, W4, b4):
    raise NotImplementedError("write your pallas kernel here")


# speedup vs baseline: 2.5709x; 2.5709x over previous
import functools

import jax
import jax.numpy as jnp
from jax import lax
from jax.experimental import pallas as pl
from jax.experimental.pallas import tpu as pltpu
from jax.experimental.pallas import tpu_sc as plsc

SC_CORES = 2
SC_SUBCORES = 16
CH = 80


def _sc_mesh():
    return plsc.VectorSubcoreMesh(
        core_axis_name="c", subcore_axis_name="s",
        num_cores=SC_CORES, num_subcores=SC_SUBCORES)


def _sc_counts(so_cat, ones128, z128, n):
    e = so_cat.shape[0] // 2
    epw = e // SC_SUBCORES
    nch = epw // CH
    stripe = n // SC_SUBCORES

    @functools.partial(
        pl.kernel,
        out_type=jax.ShapeDtypeStruct((SC_CORES, n, 128), jnp.float32),
        mesh=_sc_mesh(),
        scratch_types=[
            pltpu.VMEM((CH,), jnp.int32),
            pltpu.VMEM((CH, 128), jnp.float32),
            pltpu.VMEM_SHARED((n, 128), jnp.float32),
        ],
    )
    def k(so_hbm, ones_hbm, z_hbm, out_hbm, idx_v, ones_v, acc_sh):
        cid = lax.axis_index("c")
        sid = lax.axis_index("s")

        pltpu.sync_copy(ones_hbm, ones_v)
        pltpu.sync_copy(z_hbm.at[pl.ds(sid * stripe, stripe)],
                        acc_sh.at[pl.ds(sid * stripe, stripe)])
        plsc.subcore_barrier()

        @pl.loop(0, nch)
        def _(j):
            base = cid * e + sid * epw + j * CH
            pltpu.sync_copy(so_hbm.at[pl.ds(base, CH)], idx_v)
            pltpu.sync_copy(ones_v, acc_sh.at[idx_v], add=True)

        plsc.subcore_barrier()
        pltpu.sync_copy(acc_sh.at[pl.ds(sid * stripe, stripe)],
                        out_hbm.at[cid, pl.ds(sid * stripe, stripe)])

    return k(so_cat, ones128, z128)


def _sc_gather(obj_vecs, so_cat):
    e = so_cat.shape[0] // 2
    d = obj_vecs.shape[1]
    epw = e // SC_SUBCORES
    nch = epw // CH

    @functools.partial(
        pl.kernel,
        out_type=jax.ShapeDtypeStruct((SC_CORES, e, d), jnp.float32),
        mesh=_sc_mesh(),
        scratch_types=[
            pltpu.VMEM((CH,), jnp.int32),
            pltpu.VMEM((CH, d), jnp.float32),
            pltpu.SemaphoreType.DMA,
        ],
    )
    def k(vec_hbm, so_hbm, out_hbm, idx_v, rows_v, sem):
        cid = lax.axis_index("c")
        sid = lax.axis_index("s")

        @pl.loop(0, nch)
        def _(j):
            base = sid * epw + j * CH
            pltpu.sync_copy(so_hbm.at[pl.ds(cid * e + base, CH)], idx_v)
            pltpu.async_copy(vec_hbm.at[idx_v], rows_v, sem).wait()
            pltpu.sync_copy(rows_v, out_hbm.at[cid, pl.ds(base, CH)])

    return k(obj_vecs, so_cat)


def _sc_scatter(ns, no, so_cat, z128, n):
    e = so_cat.shape[0] // 2
    epw = e // SC_SUBCORES
    nch = epw // CH
    stripe = n // SC_SUBCORES

    @functools.partial(
        pl.kernel,
        out_type=jax.ShapeDtypeStruct((SC_CORES, n, 128), jnp.float32),
        mesh=_sc_mesh(),
        scratch_types=[
            pltpu.VMEM((CH,), jnp.int32),
            pltpu.VMEM((CH, 128), jnp.float32),
            pltpu.VMEM_SHARED((n, 128), jnp.float32),
        ],
    )
    def k(ns_hbm, no_hbm, so_hbm, z_hbm, out_hbm, idx_v, rows_v, acc_sh):
        cid = lax.axis_index("c")
        sid = lax.axis_index("s")

        pltpu.sync_copy(z_hbm.at[pl.ds(sid * stripe, stripe)],
                        acc_sh.at[pl.ds(sid * stripe, stripe)])
        plsc.subcore_barrier()

        @pl.loop(0, nch)
        def _(j):
            base = sid * epw + j * CH
            pltpu.sync_copy(so_hbm.at[pl.ds(base, CH)], idx_v)
            pltpu.sync_copy(ns_hbm.at[cid, pl.ds(base, CH)], rows_v)
            pltpu.sync_copy(rows_v, acc_sh.at[idx_v], add=True)
            pltpu.sync_copy(so_hbm.at[pl.ds(e + base, CH)], idx_v)
            pltpu.sync_copy(no_hbm.at[cid, pl.ds(base, CH)], rows_v)
            pltpu.sync_copy(rows_v, acc_sh.at[idx_v], add=True)

        plsc.subcore_barrier()
        pltpu.sync_copy(acc_sh.at[pl.ds(sid * stripe, stripe)],
                        out_hbm.at[cid, pl.ds(sid * stripe, stripe)])

    return k(ns, no, so_cat, z128)


def _tc_embed(idx3, table_pad, blk):
    nb = idx3.shape[0]
    d = table_pad.shape[1]

    def body(idx_ref, tab_ref, out_ref):
        idx = idx_ref[0, 0, :]
        oh = (idx[:, None] == lax.broadcasted_iota(jnp.int32, (blk, 128), 1))
        out_ref[...] = jnp.dot(oh.astype(jnp.float32), tab_ref[...],
                               preferred_element_type=jnp.float32)

    return pl.pallas_call(
        body,
        grid=(nb,),
        in_specs=[
            pl.BlockSpec((1, 1, blk), lambda i: (i, 0, 0)),
            pl.BlockSpec((128, d), lambda i: (0, 0)),
        ],
        out_specs=pl.BlockSpec((blk, d), lambda i: (i, 0)),
        out_shape=jax.ShapeDtypeStruct((nb * blk, d), jnp.float32),
    )(idx3, table_pad)


def _tc_edge(curso, pred_vecs, w1, b1, w2, b2, blk):
    e = pred_vecs.shape[0]
    d = pred_vecs.shape[1]
    h = w1.shape[1]
    nb = e // blk

    def body(cu_ref, pv_ref, w1_ref, b1_ref, w2_ref, b2_ref,
             ns_ref, np_ref, no_ref):
        t = jnp.concatenate([cu_ref[0], pv_ref[...], cu_ref[1]], axis=1)
        hh = jnp.dot(t, w1_ref[...], preferred_element_type=jnp.float32)
        hh = jnp.maximum(hh + b1_ref[...], 0.0)
        out = jnp.dot(hh, w2_ref[...], preferred_element_type=jnp.float32)
        out = out + b2_ref[...]
        ns_ref[0, ...] = out[:, 0:128]
        ns_ref[1, ...] = out[:, 128:256]
        np_ref[...] = out[:, 256:384]
        no_ref[0, ...] = out[:, 384:512]
        no_ref[1, ...] = out[:, 512:640]

    return pl.pallas_call(
        body,
        grid=(nb,),
        in_specs=[
            pl.BlockSpec((2, blk, d), lambda i: (0, i, 0)),
            pl.BlockSpec((blk, d), lambda i: (i, 0)),
            pl.BlockSpec((3 * d, h), lambda i: (0, 0)),
            pl.BlockSpec((1, h), lambda i: (0, 0)),
            pl.BlockSpec((h, 2 * h + d), lambda i: (0, 0)),
            pl.BlockSpec((1, 2 * h + d), lambda i: (0, 0)),
        ],
        out_specs=[
            pl.BlockSpec((2, blk, 128), lambda i: (0, i, 0)),
            pl.BlockSpec((blk, d), lambda i: (i, 0)),
            pl.BlockSpec((2, blk, 128), lambda i: (0, i, 0)),
        ],
        out_shape=[
            jax.ShapeDtypeStruct((2, e, 128), jnp.float32),
            jax.ShapeDtypeStruct((e, d), jnp.float32),
            jax.ShapeDtypeStruct((2, e, 128), jnp.float32),
        ],
    )(curso, pred_vecs, w1, b1, w2, b2)


def _tc_node(pooled2, counts2, w3, b3, w4, b4, blk):
    n = pooled2.shape[1]
    h = w3.shape[0]
    d = w4.shape[1]
    nb = n // blk

    def body(p_ref, c_ref, w3_ref, b3_ref, w4_ref, b4_ref, out_ref):
        pooled = jnp.concatenate([p_ref[0], p_ref[1]], axis=1)
        cnt = c_ref[0, :, 0:1] + c_ref[1, :, 0:1]
        pooled = pooled / jnp.maximum(cnt, 1.0)
        h2 = jnp.dot(pooled, w3_ref[...], preferred_element_type=jnp.float32)
        h2 = jnp.maximum(h2 + b3_ref[...], 0.0)
        out = jnp.dot(h2, w4_ref[...], preferred_element_type=jnp.float32)
        out_ref[...] = out + b4_ref[...]

    return pl.pallas_call(
        body,
        grid=(nb,),
        in_specs=[
            pl.BlockSpec((2, blk, 128), lambda i: (0, i, 0)),
            pl.BlockSpec((2, blk, 128), lambda i: (0, i, 0)),
            pl.BlockSpec((h, h), lambda i: (0, 0)),
            pl.BlockSpec((1, h), lambda i: (0, 0)),
            pl.BlockSpec((h, d), lambda i: (0, 0)),
            pl.BlockSpec((1, d), lambda i: (0, 0)),
        ],
        out_specs=pl.BlockSpec((blk, d), lambda i: (i, 0)),
        out_shape=jax.ShapeDtypeStruct((n, d), jnp.float32),
    )(pooled2, counts2, w3, b3, w4, b4)


def kernel(objs, triples, obj_table, pred_table, W1, b1, W2, b2, W3, b3, W4, b4):
    n = objs.shape[0]
    e = triples.shape[0]
    d = obj_table.shape[1]
    num_layers = W1.shape[0]

    s = triples[:, 0]
    p = triples[:, 1]
    o = triples[:, 2]
    so_cat = jnp.concatenate([s, o])

    obj_pad = jnp.zeros((128, d), jnp.float32).at[:obj_table.shape[0]].set(obj_table)
    pred_pad = jnp.zeros((128, d), jnp.float32).at[:pred_table.shape[0]].set(pred_table)
    npad = ((n + 16 * 8 - 1) // (16 * 8)) * (16 * 8)
    ones128 = jnp.ones((CH, 128), jnp.float32)
    z128 = jnp.zeros((npad, 128), jnp.float32)

    nblk = 2000
    eblk = 2000
    obj_embed = _tc_embed(objs.reshape(n // nblk, 1, nblk), obj_pad, nblk)
    pred_embed = _tc_embed(p.reshape(e // eblk, 1, eblk), pred_pad, eblk)

    counts2 = _sc_counts(so_cat, ones128, z128, npad)[:, :n, :]

    obj_vecs = obj_embed
    pred_vecs = pred_embed
    for i in range(num_layers):
        curso = _sc_gather(obj_vecs, so_cat)
        ns, new_p, no = _tc_edge(curso, pred_vecs, W1[i],
                                 b1[i].reshape(1, -1), W2[i],
                                 b2[i].reshape(1, -1), eblk)
        pooled2 = _sc_scatter(ns, no, so_cat, z128, npad)[:, :n, :]
        obj_vecs = _tc_node(pooled2, counts2, W3[i], b3[i].reshape(1, -1),
                            W4[i], b4[i].reshape(1, -1), nblk)
        pred_vecs = new_p

    return (obj_embed, pred_embed, obj_vecs, pred_vecs)


# speedup vs baseline: 4.3570x; 1.6947x over previous
import functools

import jax
import jax.numpy as jnp
from jax import lax
from jax.experimental import pallas as pl
from jax.experimental.pallas import tpu as pltpu
from jax.experimental.pallas import tpu_sc as plsc

SC_CORES = 2
SC_SUBCORES = 16
CH = 80


def _sc_mesh():
    return plsc.VectorSubcoreMesh(
        core_axis_name="c", subcore_axis_name="s",
        num_cores=SC_CORES, num_subcores=SC_SUBCORES)


def _sc_counts(so_cat, ones128, z128, n):
    e = so_cat.shape[0] // 2
    epw = e // SC_SUBCORES
    nch = epw // CH
    stripe = n // SC_SUBCORES

    @functools.partial(
        pl.kernel,
        out_type=jax.ShapeDtypeStruct((SC_CORES, n, 128), jnp.float32),
        mesh=_sc_mesh(),
        scratch_types=[
            pltpu.VMEM((CH,), jnp.int32),
            pltpu.VMEM((CH, 128), jnp.float32),
            pltpu.VMEM_SHARED((n, 128), jnp.float32),
        ],
    )
    def k(so_hbm, ones_hbm, z_hbm, out_hbm, idx_v, ones_v, acc_sh):
        cid = lax.axis_index("c")
        sid = lax.axis_index("s")

        pltpu.sync_copy(ones_hbm, ones_v)
        pltpu.sync_copy(z_hbm.at[pl.ds(sid * stripe, stripe)],
                        acc_sh.at[pl.ds(sid * stripe, stripe)])
        plsc.subcore_barrier()

        @pl.loop(0, nch)
        def _(j):
            base = cid * e + sid * epw + j * CH
            pltpu.sync_copy(so_hbm.at[pl.ds(base, CH)], idx_v)
            pltpu.sync_copy(ones_v, acc_sh.at[idx_v], add=True)

        plsc.subcore_barrier()
        pltpu.sync_copy(acc_sh.at[pl.ds(sid * stripe, stripe)],
                        out_hbm.at[cid, pl.ds(sid * stripe, stripe)])

    return k(so_cat, ones128, z128)


def _sc_gather(obj_vecs, so_cat):
    e = so_cat.shape[0] // 2
    d = obj_vecs.shape[1]
    epw = e // SC_SUBCORES
    nch = epw // CH
    nch2 = nch // 2

    @functools.partial(
        pl.kernel,
        out_type=jax.ShapeDtypeStruct((SC_CORES, e, d), jnp.float32),
        mesh=_sc_mesh(),
        scratch_types=[
            pltpu.VMEM((epw,), jnp.int32),
            pltpu.VMEM((2, CH, d), jnp.float32),
            pltpu.SemaphoreType.DMA,
            pltpu.SemaphoreType.DMA,
            pltpu.SemaphoreType.DMA,
            pltpu.SemaphoreType.DMA,
        ],
    )
    def k(vec_hbm, so_hbm, out_hbm, idx_v, rows_v, sg0, sg1, so0, so1):
        cid = lax.axis_index("c")
        sid = lax.axis_index("s")
        rb = sid * epw

        pltpu.sync_copy(so_hbm.at[pl.ds(cid * e + rb, epw)], idx_v)

        def start_g(j, b, sem):
            pltpu.async_copy(vec_hbm.at[idx_v.at[pl.ds(j * CH, CH)]],
                             rows_v.at[b], sem)

        def wait_g(b, sem):
            pltpu.make_async_copy(vec_hbm.at[idx_v.at[pl.ds(0, CH)]],
                                  rows_v.at[b], sem).wait()

        def start_o(j, b, sem):
            pltpu.async_copy(rows_v.at[b],
                             out_hbm.at[cid, pl.ds(rb + j * CH, CH)], sem)

        def wait_o(b, sem):
            pltpu.make_async_copy(rows_v.at[b],
                                  out_hbm.at[cid, pl.ds(0, CH)], sem).wait()

        start_g(0, 0, sg0)

        @pl.loop(0, nch2)
        def _(t):
            j0 = 2 * t

            @pl.when(t > 0)
            def _():
                wait_o(1, so1)

            start_g(j0 + 1, 1, sg1)
            wait_g(0, sg0)
            start_o(j0, 0, so0)
            wait_g(1, sg1)
            wait_o(0, so0)

            @pl.when(t < nch2 - 1)
            def _():
                start_g(j0 + 2, 0, sg0)

            start_o(j0 + 1, 1, so1)

        wait_o(1, so1)

    return k(obj_vecs, so_cat)


def _sc_scatter(nsno, so_il, z128, n):
    e2 = so_il.shape[0]
    epw = e2 // SC_SUBCORES
    nch = epw // CH
    nch2 = nch // 2
    stripe = n // SC_SUBCORES

    @functools.partial(
        pl.kernel,
        out_type=jax.ShapeDtypeStruct((SC_CORES, n, 128), jnp.float32),
        mesh=_sc_mesh(),
        scratch_types=[
            pltpu.VMEM((2, CH), jnp.int32),
            pltpu.VMEM((2, CH, 128), jnp.float32),
            pltpu.VMEM_SHARED((n, 128), jnp.float32),
            pltpu.SemaphoreType.DMA,
            pltpu.SemaphoreType.DMA,
            pltpu.SemaphoreType.DMA,
            pltpu.SemaphoreType.DMA,
        ],
    )
    def k(v_hbm, i_hbm, z_hbm, out_hbm, idx_v, val_v, acc_sh,
          si0, si1, sa0, sa1):
        cid = lax.axis_index("c")
        sid = lax.axis_index("s")
        rb = sid * epw

        pltpu.sync_copy(z_hbm.at[pl.ds(sid * stripe, stripe)],
                        acc_sh.at[pl.ds(sid * stripe, stripe)])
        plsc.subcore_barrier()

        def start_in(j, b, sem):
            pltpu.async_copy(i_hbm.at[pl.ds(rb + j * CH, CH)],
                             idx_v.at[b], sem)
            pltpu.async_copy(v_hbm.at[cid, pl.ds(rb + j * CH, CH)],
                             val_v.at[b], sem)

        def wait_in(b, sem):
            pltpu.make_async_copy(i_hbm.at[pl.ds(0, CH)],
                                  idx_v.at[b], sem).wait()
            pltpu.make_async_copy(v_hbm.at[cid, pl.ds(0, CH)],
                                  val_v.at[b], sem).wait()

        def start_add(b, sem):
            pltpu.async_copy(val_v.at[b], acc_sh.at[idx_v.at[b]], sem,
                             add=True)

        def wait_add(b, sem):
            pltpu.make_async_copy(val_v.at[b],
                                  acc_sh.at[idx_v.at[b]], sem).wait()

        start_in(0, 0, si0)

        @pl.loop(0, nch2)
        def _(t):
            j0 = 2 * t

            @pl.when(t > 0)
            def _():
                wait_add(1, sa1)

            start_in(j0 + 1, 1, si1)
            wait_in(0, si0)
            start_add(0, sa0)
            wait_in(1, si1)
            wait_add(0, sa0)

            @pl.when(t < nch2 - 1)
            def _():
                start_in(j0 + 2, 0, si0)

            start_add(1, sa1)

        wait_add(1, sa1)
        plsc.subcore_barrier()
        pltpu.sync_copy(acc_sh.at[pl.ds(sid * stripe, stripe)],
                        out_hbm.at[cid, pl.ds(sid * stripe, stripe)])

    return k(nsno, so_il, z128)


def _tc_embed(idx3, table_pad, blk):
    nb = idx3.shape[0]
    d = table_pad.shape[1]

    def body(idx_ref, tab_ref, out_ref):
        idx = idx_ref[0, 0, :]
        oh = (idx[:, None] == lax.broadcasted_iota(jnp.int32, (blk, 128), 1))
        out_ref[...] = jnp.dot(oh.astype(jnp.float32), tab_ref[...],
                               preferred_element_type=jnp.float32)

    return pl.pallas_call(
        body,
        grid=(nb,),
        in_specs=[
            pl.BlockSpec((1, 1, blk), lambda i: (i, 0, 0)),
            pl.BlockSpec((128, d), lambda i: (0, 0)),
        ],
        out_specs=pl.BlockSpec((blk, d), lambda i: (i, 0)),
        out_shape=jax.ShapeDtypeStruct((nb * blk, d), jnp.float32),
    )(idx3, table_pad)


def _tc_edge(curso, pred_vecs, w1, b1, w2, b2, blk):
    e = pred_vecs.shape[0]
    d = pred_vecs.shape[1]
    h = w1.shape[1]
    nb = e // blk

    def body(cu_ref, pv_ref, w1_ref, b1_ref, w2_ref, b2_ref,
             nsno_ref, np_ref):
        t = jnp.concatenate([cu_ref[0], pv_ref[...], cu_ref[1]], axis=1)
        hh = jnp.dot(t, w1_ref[...], preferred_element_type=jnp.float32)
        hh = jnp.maximum(hh + b1_ref[...], 0.0)
        out = jnp.dot(hh, w2_ref[...], preferred_element_type=jnp.float32)
        out = out + b2_ref[...]
        nsno_ref[0, 0, 0] = out[:, 0:128]
        nsno_ref[1, 0, 0] = out[:, 128:256]
        np_ref[...] = out[:, 256:384]
        nsno_ref[0, 0, 1] = out[:, 384:512]
        nsno_ref[1, 0, 1] = out[:, 512:640]

    return pl.pallas_call(
        body,
        grid=(nb,),
        in_specs=[
            pl.BlockSpec((2, blk, d), lambda i: (0, i, 0)),
            pl.BlockSpec((blk, d), lambda i: (i, 0)),
            pl.BlockSpec((3 * d, h), lambda i: (0, 0)),
            pl.BlockSpec((1, h), lambda i: (0, 0)),
            pl.BlockSpec((h, 2 * h + d), lambda i: (0, 0)),
            pl.BlockSpec((1, 2 * h + d), lambda i: (0, 0)),
        ],
        out_specs=[
            pl.BlockSpec((2, 1, 2, blk, 128), lambda i: (0, i, 0, 0, 0)),
            pl.BlockSpec((blk, d), lambda i: (i, 0)),
        ],
        out_shape=[
            jax.ShapeDtypeStruct((2, nb, 2, blk, 128), jnp.float32),
            jax.ShapeDtypeStruct((e, d), jnp.float32),
        ],
    )(curso, pred_vecs, w1, b1, w2, b2)


def _tc_node(pooled2, counts2, w3, b3, w4, b4, blk):
    n = pooled2.shape[1]
    h = w3.shape[0]
    d = w4.shape[1]
    nb = n // blk

    def body(p_ref, c_ref, w3_ref, b3_ref, w4_ref, b4_ref, out_ref):
        pooled = jnp.concatenate([p_ref[0], p_ref[1]], axis=1)
        cnt = c_ref[0, :, 0:1] + c_ref[1, :, 0:1]
        pooled = pooled / jnp.maximum(cnt, 1.0)
        h2 = jnp.dot(pooled, w3_ref[...], preferred_element_type=jnp.float32)
        h2 = jnp.maximum(h2 + b3_ref[...], 0.0)
        out = jnp.dot(h2, w4_ref[...], preferred_element_type=jnp.float32)
        out_ref[...] = out + b4_ref[...]

    return pl.pallas_call(
        body,
        grid=(nb,),
        in_specs=[
            pl.BlockSpec((2, blk, 128), lambda i: (0, i, 0)),
            pl.BlockSpec((2, blk, 128), lambda i: (0, i, 0)),
            pl.BlockSpec((h, h), lambda i: (0, 0)),
            pl.BlockSpec((1, h), lambda i: (0, 0)),
            pl.BlockSpec((h, d), lambda i: (0, 0)),
            pl.BlockSpec((1, d), lambda i: (0, 0)),
        ],
        out_specs=pl.BlockSpec((blk, d), lambda i: (i, 0)),
        out_shape=jax.ShapeDtypeStruct((n, d), jnp.float32),
    )(pooled2, counts2, w3, b3, w4, b4)


def kernel(objs, triples, obj_table, pred_table, W1, b1, W2, b2, W3, b3, W4, b4):
    n = objs.shape[0]
    e = triples.shape[0]
    d = obj_table.shape[1]
    num_layers = W1.shape[0]

    s = triples[:, 0]
    p = triples[:, 1]
    o = triples[:, 2]
    so_cat = jnp.concatenate([s, o])
    eblk = 2000
    nblk = 2000
    nb = e // eblk
    so_il = jnp.stack([s.reshape(nb, eblk), o.reshape(nb, eblk)],
                      axis=1).reshape(-1)

    obj_pad = jnp.zeros((128, d), jnp.float32).at[:obj_table.shape[0]].set(obj_table)
    pred_pad = jnp.zeros((128, d), jnp.float32).at[:pred_table.shape[0]].set(pred_table)
    npad = ((n + 16 * 8 - 1) // (16 * 8)) * (16 * 8)
    ones128 = jnp.ones((CH, 128), jnp.float32)
    z128 = jnp.zeros((npad, 128), jnp.float32)

    obj_embed = _tc_embed(objs.reshape(n // nblk, 1, nblk), obj_pad, nblk)
    pred_embed = _tc_embed(p.reshape(e // eblk, 1, eblk), pred_pad, eblk)

    counts2 = _sc_counts(so_cat, ones128, z128, npad)[:, :n, :]

    obj_vecs = obj_embed
    pred_vecs = pred_embed
    for i in range(num_layers):
        curso = _sc_gather(obj_vecs, so_cat)
        nsno, new_p = _tc_edge(curso, pred_vecs, W1[i],
                               b1[i].reshape(1, -1), W2[i],
                               b2[i].reshape(1, -1), eblk)
        pooled2 = _sc_scatter(nsno.reshape(2, 2 * e, 128), so_il,
                              z128, npad)[:, :n, :]
        obj_vecs = _tc_node(pooled2, counts2, W3[i], b3[i].reshape(1, -1),
                            W4[i], b4[i].reshape(1, -1), nblk)
        pred_vecs = new_p

    return (obj_embed, pred_embed, obj_vecs, pred_vecs)


# speedup vs baseline: 4.6099x; 1.0581x over previous
import functools

import jax
import jax.numpy as jnp
from jax import lax
from jax.experimental import pallas as pl
from jax.experimental.pallas import tpu as pltpu
from jax.experimental.pallas import tpu_sc as plsc

SC_CORES = 2
SC_SUBCORES = 16
CH = 80


def _sc_mesh():
    return plsc.VectorSubcoreMesh(
        core_axis_name="c", subcore_axis_name="s",
        num_cores=SC_CORES, num_subcores=SC_SUBCORES)


def _sc_counts(so_cat, ones128, z128, n):
    e = so_cat.shape[0] // 2
    epw = e // SC_SUBCORES
    nch = epw // CH
    stripe = n // SC_SUBCORES

    @functools.partial(
        pl.kernel,
        out_type=jax.ShapeDtypeStruct((SC_CORES, n, 128), jnp.float32),
        mesh=_sc_mesh(),
        scratch_types=[
            pltpu.VMEM((CH,), jnp.int32),
            pltpu.VMEM((CH, 128), jnp.float32),
            pltpu.VMEM_SHARED((n, 128), jnp.float32),
        ],
    )
    def k(so_hbm, ones_hbm, z_hbm, out_hbm, idx_v, ones_v, acc_sh):
        cid = lax.axis_index("c")
        sid = lax.axis_index("s")

        pltpu.sync_copy(ones_hbm, ones_v)
        pltpu.sync_copy(z_hbm.at[pl.ds(sid * stripe, stripe)],
                        acc_sh.at[pl.ds(sid * stripe, stripe)])
        plsc.subcore_barrier()

        @pl.loop(0, nch)
        def _(j):
            base = cid * e + sid * epw + j * CH
            pltpu.sync_copy(so_hbm.at[pl.ds(base, CH)], idx_v)
            pltpu.sync_copy(ones_v, acc_sh.at[idx_v], add=True)

        plsc.subcore_barrier()
        pltpu.sync_copy(acc_sh.at[pl.ds(sid * stripe, stripe)],
                        out_hbm.at[cid, pl.ds(sid * stripe, stripe)])

    return k(so_cat, ones128, z128)


def _sc_gather(obj_vecs, so_cat):
    e = so_cat.shape[0] // 2
    d = obj_vecs.shape[1]
    epw = e // SC_SUBCORES
    nch = epw // CH
    npairs = nch // 2
    has_tail = bool(nch % 2)

    @functools.partial(
        pl.kernel,
        out_type=jax.ShapeDtypeStruct((SC_CORES, e, d), jnp.float32),
        mesh=_sc_mesh(),
        scratch_types=[
            pltpu.VMEM((epw,), jnp.int32),
            pltpu.VMEM((2, CH, d), jnp.float32),
            pltpu.SemaphoreType.DMA,
            pltpu.SemaphoreType.DMA,
            pltpu.SemaphoreType.DMA,
            pltpu.SemaphoreType.DMA,
        ],
    )
    def k(vec_hbm, so_hbm, out_hbm, idx_v, rows_v, sg0, sg1, so0, so1):
        cid = lax.axis_index("c")
        sid = lax.axis_index("s")
        rb = sid * epw

        pltpu.sync_copy(so_hbm.at[pl.ds(cid * e + rb, epw)], idx_v)

        def start_g(j, b, sem):
            pltpu.async_copy(vec_hbm.at[idx_v.at[pl.ds(j * CH, CH)]],
                             rows_v.at[b], sem)

        def wait_g(b, sem):
            pltpu.make_async_copy(vec_hbm.at[idx_v.at[pl.ds(0, CH)]],
                                  rows_v.at[b], sem).wait()

        def start_o(j, b, sem):
            pltpu.async_copy(rows_v.at[b],
                             out_hbm.at[cid, pl.ds(rb + j * CH, CH)], sem)

        def wait_o(b, sem):
            pltpu.make_async_copy(rows_v.at[b],
                                  out_hbm.at[cid, pl.ds(0, CH)], sem).wait()

        start_g(0, 0, sg0)

        @pl.loop(0, npairs)
        def _(t):
            j0 = 2 * t

            @pl.when(t > 0)
            def _():
                wait_o(1, so1)

            start_g(j0 + 1, 1, sg1)
            wait_g(0, sg0)
            start_o(j0, 0, so0)
            wait_g(1, sg1)
            wait_o(0, so0)

            if has_tail:
                start_g(j0 + 2, 0, sg0)
            else:
                @pl.when(t < npairs - 1)
                def _():
                    start_g(j0 + 2, 0, sg0)

            start_o(j0 + 1, 1, so1)

        if has_tail:
            wait_g(0, sg0)
            start_o(nch - 1, 0, so0)
            wait_o(0, so0)
        wait_o(1, so1)

    return k(obj_vecs, so_cat)


def _sc_scatter(nsno, so_il, z128, n):
    e2 = so_il.shape[0]
    epw = e2 // SC_SUBCORES
    nch = epw // CH
    nch2 = nch // 2
    stripe = n // SC_SUBCORES

    @functools.partial(
        pl.kernel,
        out_type=jax.ShapeDtypeStruct((SC_CORES, n, 128), jnp.float32),
        mesh=_sc_mesh(),
        scratch_types=[
            pltpu.VMEM((2, CH), jnp.int32),
            pltpu.VMEM((2, CH, 128), jnp.float32),
            pltpu.VMEM_SHARED((n, 128), jnp.float32),
            pltpu.SemaphoreType.DMA,
            pltpu.SemaphoreType.DMA,
            pltpu.SemaphoreType.DMA,
            pltpu.SemaphoreType.DMA,
        ],
    )
    def k(v_hbm, i_hbm, z_hbm, out_hbm, idx_v, val_v, acc_sh,
          si0, si1, sa0, sa1):
        cid = lax.axis_index("c")
        sid = lax.axis_index("s")
        rb = sid * epw

        pltpu.sync_copy(z_hbm.at[pl.ds(sid * stripe, stripe)],
                        acc_sh.at[pl.ds(sid * stripe, stripe)])
        plsc.subcore_barrier()

        def start_in(j, b, sem):
            pltpu.async_copy(i_hbm.at[pl.ds(rb + j * CH, CH)],
                             idx_v.at[b], sem)
            pltpu.async_copy(v_hbm.at[cid, pl.ds(rb + j * CH, CH)],
                             val_v.at[b], sem)

        def wait_in(b, sem):
            pltpu.make_async_copy(i_hbm.at[pl.ds(0, CH)],
                                  idx_v.at[b], sem).wait()
            pltpu.make_async_copy(v_hbm.at[cid, pl.ds(0, CH)],
                                  val_v.at[b], sem).wait()

        def start_add(b, sem):
            pltpu.async_copy(val_v.at[b], acc_sh.at[idx_v.at[b]], sem,
                             add=True)

        def wait_add(b, sem):
            pltpu.make_async_copy(val_v.at[b],
                                  acc_sh.at[idx_v.at[b]], sem).wait()

        start_in(0, 0, si0)

        @pl.loop(0, nch2)
        def _(t):
            j0 = 2 * t

            @pl.when(t > 0)
            def _():
                wait_add(1, sa1)

            start_in(j0 + 1, 1, si1)
            wait_in(0, si0)
            start_add(0, sa0)
            wait_in(1, si1)
            wait_add(0, sa0)

            @pl.when(t < nch2 - 1)
            def _():
                start_in(j0 + 2, 0, si0)

            start_add(1, sa1)

        wait_add(1, sa1)
        plsc.subcore_barrier()
        pltpu.sync_copy(acc_sh.at[pl.ds(sid * stripe, stripe)],
                        out_hbm.at[cid, pl.ds(sid * stripe, stripe)])

    return k(nsno, so_il, z128)


def _tc_embed(idx3, table_pad, blk):
    nb = idx3.shape[0]
    d = table_pad.shape[1]

    def body(idx_ref, tab_ref, out_ref):
        idx = idx_ref[0, 0, :]
        oh = (idx[:, None] == lax.broadcasted_iota(jnp.int32, (blk, 128), 1))
        out_ref[...] = jnp.dot(oh.astype(jnp.float32), tab_ref[...],
                               preferred_element_type=jnp.float32)

    return pl.pallas_call(
        body,
        grid=(nb,),
        in_specs=[
            pl.BlockSpec((1, 1, blk), lambda i: (i, 0, 0)),
            pl.BlockSpec((128, d), lambda i: (0, 0)),
        ],
        out_specs=pl.BlockSpec((blk, d), lambda i: (i, 0)),
        out_shape=jax.ShapeDtypeStruct((nb * blk, d), jnp.float32),
    )(idx3, table_pad)


def _tc_edge(curso, pred_vecs, w1, b1, w2, b2, blk, pv_off):
    e = curso.shape[1]
    d = pred_vecs.shape[1]
    h = w1.shape[1]
    nb = e // blk

    def body(cu_ref, pv_ref, w1_ref, b1_ref, w2_ref, b2_ref,
             nsno_ref, np_ref):
        t = jnp.concatenate([cu_ref[0], pv_ref[...], cu_ref[1]], axis=1)
        hh = jnp.dot(t, w1_ref[...], preferred_element_type=jnp.float32)
        hh = jnp.maximum(hh + b1_ref[...], 0.0)
        out = jnp.dot(hh, w2_ref[...], preferred_element_type=jnp.float32)
        out = out + b2_ref[...]
        nsno_ref[0, 0, 0] = out[:, 0:128]
        nsno_ref[1, 0, 0] = out[:, 128:256]
        np_ref[...] = out[:, 256:384]
        nsno_ref[0, 0, 1] = out[:, 384:512]
        nsno_ref[1, 0, 1] = out[:, 512:640]

    return pl.pallas_call(
        body,
        grid=(nb,),
        in_specs=[
            pl.BlockSpec((2, blk, d), lambda i: (0, i, 0)),
            pl.BlockSpec((blk, d), lambda i: (i + pv_off, 0)),
            pl.BlockSpec((3 * d, h), lambda i: (0, 0)),
            pl.BlockSpec((1, h), lambda i: (0, 0)),
            pl.BlockSpec((h, 2 * h + d), lambda i: (0, 0)),
            pl.BlockSpec((1, 2 * h + d), lambda i: (0, 0)),
        ],
        out_specs=[
            pl.BlockSpec((2, 1, 2, blk, 128), lambda i: (0, i, 0, 0, 0)),
            pl.BlockSpec((blk, d), lambda i: (i, 0)),
        ],
        out_shape=[
            jax.ShapeDtypeStruct((2, nb, 2, blk, 128), jnp.float32),
            jax.ShapeDtypeStruct((e, d), jnp.float32),
        ],
    )(curso, pred_vecs, w1, b1, w2, b2)


def _tc_node(pa, pb, counts2, w3, b3, w4, b4, n, blk):
    h = w3.shape[0]
    d = w4.shape[1]
    nb = n // blk

    def body(pa_ref, pb_ref, c_ref, w3_ref, b3_ref, w4_ref, b4_ref, out_ref):
        pooled = (jnp.concatenate([pa_ref[0], pa_ref[1]], axis=1)
                  + jnp.concatenate([pb_ref[0], pb_ref[1]], axis=1))
        cnt = c_ref[0, :, 0:1] + c_ref[1, :, 0:1]
        pooled = pooled / jnp.maximum(cnt, 1.0)
        h2 = jnp.dot(pooled, w3_ref[...], preferred_element_type=jnp.float32)
        h2 = jnp.maximum(h2 + b3_ref[...], 0.0)
        out = jnp.dot(h2, w4_ref[...], preferred_element_type=jnp.float32)
        out_ref[...] = out + b4_ref[...]

    return pl.pallas_call(
        body,
        grid=(nb,),
        in_specs=[
            pl.BlockSpec((2, blk, 128), lambda i: (0, i, 0)),
            pl.BlockSpec((2, blk, 128), lambda i: (0, i, 0)),
            pl.BlockSpec((2, blk, 128), lambda i: (0, i, 0)),
            pl.BlockSpec((h, h), lambda i: (0, 0)),
            pl.BlockSpec((1, h), lambda i: (0, 0)),
            pl.BlockSpec((h, d), lambda i: (0, 0)),
            pl.BlockSpec((1, d), lambda i: (0, 0)),
        ],
        out_specs=pl.BlockSpec((blk, d), lambda i: (i, 0)),
        out_shape=jax.ShapeDtypeStruct((n, d), jnp.float32),
    )(pa, pb, counts2, w3, b3, w4, b4)


def kernel(objs, triples, obj_table, pred_table, W1, b1, W2, b2, W3, b3, W4, b4):
    n = objs.shape[0]
    e = triples.shape[0]
    d = obj_table.shape[1]
    num_layers = W1.shape[0]

    s = triples[:, 0]
    p = triples[:, 1]
    o = triples[:, 2]
    so_cat = jnp.concatenate([s, o])
    eblk = 2000
    nblk = 2000
    nb = e // eblk
    half = e // 2
    nbh = nb // 2
    so_a = jnp.concatenate([s[:half], o[:half]])
    so_b = jnp.concatenate([s[half:], o[half:]])
    so_il = jnp.stack([s.reshape(nb, eblk), o.reshape(nb, eblk)],
                      axis=1).reshape(-1)
    so_il_a = so_il[:e]
    so_il_b = so_il[e:]

    obj_pad = jnp.zeros((128, d), jnp.float32).at[:obj_table.shape[0]].set(obj_table)
    pred_pad = jnp.zeros((128, d), jnp.float32).at[:pred_table.shape[0]].set(pred_table)
    npad = ((n + 16 * 8 - 1) // (16 * 8)) * (16 * 8)
    ones128 = jnp.ones((CH, 128), jnp.float32)
    z128 = jnp.zeros((npad, 128), jnp.float32)

    obj_embed = _tc_embed(objs.reshape(n // nblk, 1, nblk), obj_pad, nblk)
    pred_embed = _tc_embed(p.reshape(e // eblk, 1, eblk), pred_pad, eblk)

    counts2 = _sc_counts(so_cat, ones128, z128, npad)

    obj_vecs = obj_embed
    pv_a = pred_embed[:half]
    pv_b = pred_embed[half:]
    for i in range(num_layers):
        w1 = W1[i]
        b1i = b1[i].reshape(1, -1)
        w2 = W2[i]
        b2i = b2[i].reshape(1, -1)
        curso_a = _sc_gather(obj_vecs, so_a)
        curso_b = _sc_gather(obj_vecs, so_b)
        nsno_a, np_a = _tc_edge(curso_a, pv_a, w1, b1i, w2, b2i, eblk, 0)
        pooled_a = _sc_scatter(nsno_a.reshape(2, e, 128), so_il_a, z128, npad)
        nsno_b, np_b = _tc_edge(curso_b, pv_b, w1, b1i, w2, b2i, eblk, 0)
        pooled_b = _sc_scatter(nsno_b.reshape(2, e, 128), so_il_b, z128, npad)
        obj_vecs = _tc_node(pooled_a, pooled_b, counts2, W3[i],
                            b3[i].reshape(1, -1), W4[i],
                            b4[i].reshape(1, -1), n, nblk)
        pv_a, pv_b = np_a, np_b

    return (obj_embed, pred_embed, obj_vecs, jnp.concatenate([pv_a, pv_b]))
